# baseline (speedup 1.0000x reference)
import functools

import jax
import jax.numpy as jnp
from jax import lax
from jax.experimental import pallas as pl
from jax.experimental.pallas import tpu as pltpu
from jax.experimental.pallas import tpu_sc as plsc

B = 16
L1 = 4096
MAXQ = 2048
MAXT = 512
MAXC = 1536
L2 = MAXQ + MAXT + MAXC
D = 512

CHUNK = 32
WIN = CHUNK + 8
LANES = 16
NSUB = 16
EPC = B // 2
CPT = EPC * (L2 // CHUNK) // NSUB
QCH = MAXQ // CHUNK
TCH = MAXT // CHUNK


def _mult8(x):
    return pl.multiple_of(x, 8)


def _repack_body(inputs_hbm, maski_hbm, out_hbm,
                 mrow, ring0, ring1, obuf0, obuf1, zbuf, svbuf, sumsbuf,
                 shared, rsem0, rsem1, wsem0, wsem1, msem):
    cid = lax.axis_index("c")
    sid = lax.axis_index("s")

    zvec = jnp.zeros((LANES,), jnp.float32)
    lane = jnp.arange(LANES, dtype=jnp.int32)

    @pl.when(sid < EPC)
    def _():
        bex = cid * EPC + sid
        pltpu.make_async_copy(maski_hbm.at[pl.ds(bex * L2, L2)], mrow,
                              msem).start()
        pltpu.make_async_copy(maski_hbm.at[pl.ds(bex * L2, L2)], mrow,
                              msem).wait()

        def msum(start, count):
            def sbody(i, acc):
                return acc + mrow[pl.ds(start + i * LANES, LANES)]
            acc = lax.fori_loop(0, count // LANES, sbody,
                                jnp.zeros((LANES,), jnp.int32))
            s = acc[0]
            for i in range(1, LANES):
                s = s + acc[i]
            return s

        q = msum(0, MAXQ)
        t = msum(MAXQ, MAXT)
        c = msum(MAXQ + MAXT, MAXC)
        svec = jnp.where(lane == 0, jnp.full((LANES,), q, jnp.int32),
                         jnp.where(lane == 1,
                                   jnp.full((LANES,), t, jnp.int32),
                                   jnp.full((LANES,), c, jnp.int32)))
        svbuf[pl.ds(0, LANES)] = svec
        pltpu.sync_copy(svbuf, shared.at[pl.ds(sid * LANES, LANES)])

    def zb(j, _):
        zbuf[j // (D // LANES), pl.ds((j % (D // LANES)) * LANES, LANES)] = zvec
        return 0
    lax.fori_loop(0, (CHUNK * D) // LANES, zb, 0)

    plsc.subcore_barrier()
    pltpu.sync_copy(shared, sumsbuf)

    def chunk_info(j):
        g128 = sid + NSUB * j
        bl = g128 // (L2 // CHUNK)
        g = g128 - bl * (L2 // CHUNK)
        svec = sumsbuf[pl.ds(bl * LANES, LANES)]
        q, t, c = svec[0], svec[1], svec[2]
        in_t = jnp.logical_and(g >= QCH, g < QCH + TCH)
        in_c = g >= QCH + TCH
        gg = jnp.where(in_c, g - QCH - TCH, jnp.where(in_t, g - QCH, g))
        n = jnp.where(in_c, c, jnp.where(in_t, t, q))
        src_row = jnp.where(in_c, q + t, jnp.where(in_t, q, 0))
        nv = jnp.minimum(n - gg * CHUNK, CHUNK)
        bex = cid * EPC + bl
        src = src_row + gg * CHUNK
        wstart = _mult8(jnp.minimum(src - src % 8, L1 - WIN))
        wref = inputs_hbm.at[bex, pl.ds(wstart, WIN), :]
        dref = out_hbm.at[bex, pl.ds(_mult8(g * CHUNK), CHUNK), :]
        return nv > 0, wref, src - wstart, nv, dref

    def shift_chunk(src_buf, sbase, nvalid, dst_buf):
        def crow(j, _):
            for ci in range(D // LANES):
                dst_buf[j, pl.ds(ci * LANES, LANES)] = (
                    src_buf[sbase + j, pl.ds(ci * LANES, LANES)])
            return 0
        lax.fori_loop(0, nvalid, crow, 0)

        def zrow(j, _):
            for ci in range(D // LANES):
                dst_buf[j, pl.ds(ci * LANES, LANES)] = zvec
            return 0
        lax.fori_loop(nvalid, CHUNK, zrow, 0)

    def proc(j, p, ring, obuf, rsem, wsem):
        valid, wref, sbase, nv, dref = chunk_info(j)

        @pl.when(p > 0)
        def _():
            pltpu.make_async_copy(obuf, dref, wsem).wait()

        @pl.when(valid)
        def _():
            pltpu.make_async_copy(wref, ring, rsem).wait()
            shift_chunk(ring, sbase, nv, obuf)
            pltpu.make_async_copy(obuf, dref, wsem).start()

        @pl.when(jnp.logical_not(valid))
        def _():
            pltpu.make_async_copy(zbuf, dref, wsem).start()

    def prefetch(j, ring, rsem):
        valid, wref, _, _, _ = chunk_info(j)

        @pl.when(valid)
        def _():
            pltpu.make_async_copy(wref, ring, rsem).start()

    prefetch(0, ring0, rsem0)

    def pair(p, _):
        j0 = 2 * p
        j1 = j0 + 1
        prefetch(j1, ring1, rsem1)
        proc(j0, p, ring0, obuf0, rsem0, wsem0)

        @pl.when(j0 + 2 < CPT)
        def _():
            prefetch(j0 + 2, ring0, rsem0)

        proc(j1, p, ring1, obuf1, rsem1, wsem1)
        return 0
    lax.fori_loop(0, CPT // 2, pair, 0)

    last = out_hbm.at[cid * EPC, pl.ds(0, CHUNK), :]
    pltpu.make_async_copy(obuf0, last, wsem0).wait()
    pltpu.make_async_copy(obuf1, last, wsem1).wait()


@jax.jit
def _repack(inputs, maski):
    mesh = plsc.VectorSubcoreMesh(core_axis_name="c", subcore_axis_name="s")
    k = functools.partial(
        pl.kernel,
        mesh=mesh,
        out_type=jax.ShapeDtypeStruct((B, L2, D), jnp.float32),
        scratch_types=[
            pltpu.VMEM((L2,), jnp.int32),
            pltpu.VMEM((WIN, D), jnp.float32),
            pltpu.VMEM((WIN, D), jnp.float32),
            pltpu.VMEM((CHUNK, D), jnp.float32),
            pltpu.VMEM((CHUNK, D), jnp.float32),
            pltpu.VMEM((CHUNK, D), jnp.float32),
            pltpu.VMEM((LANES,), jnp.int32),
            pltpu.VMEM((EPC * LANES,), jnp.int32),
            pltpu.VMEM_SHARED((EPC * LANES,), jnp.int32),
            pltpu.SemaphoreType.DMA,
            pltpu.SemaphoreType.DMA,
            pltpu.SemaphoreType.DMA,
            pltpu.SemaphoreType.DMA,
            pltpu.SemaphoreType.DMA,
        ],
    )(_repack_body)
    return k(inputs, maski)


def kernel(inputs, mask, mask_split):
    del mask
    outputs = _repack(inputs, mask_split.astype(jnp.int32).reshape(-1))
    return outputs, mask_split


# speedup vs baseline: 1.4044x; 1.4044x over previous
import functools

import jax
import jax.numpy as jnp
from jax import lax
from jax.experimental import pallas as pl
from jax.experimental.pallas import tpu as pltpu
from jax.experimental.pallas import tpu_sc as plsc

B = 16
L1 = 4096
MAXQ = 2048
MAXT = 512
MAXC = 1536
L2 = MAXQ + MAXT + MAXC
D = 512

CHUNK = 32
WIN = CHUNK + 8
LANES = 16
NSUB = 16
EPC = B // 2
CPT = EPC * (L2 // CHUNK) // NSUB
QCH = MAXQ // CHUNK
TCH = MAXT // CHUNK


def _mult8(x):
    return pl.multiple_of(x, 8)


def _repack_body(inputs_hbm, maski_hbm, out_hbm,
                 mrow, ring0, ring1, obuf0, obuf1, zbuf, svbuf, sumsbuf,
                 shared, rsem0, rsem1, wsem0, wsem1, msem):
    cid = lax.axis_index("c")
    sid = lax.axis_index("s")

    zvec = jnp.zeros((LANES,), jnp.float32)
    lane = jnp.arange(LANES, dtype=jnp.int32)

    @pl.when(sid < EPC)
    def _():
        bex = cid * EPC + sid
        pltpu.make_async_copy(maski_hbm.at[pl.ds(bex * L2, L2)], mrow,
                              msem).start()
        pltpu.make_async_copy(maski_hbm.at[pl.ds(bex * L2, L2)], mrow,
                              msem).wait()

        def msum(start, count):
            def sbody(i, acc):
                return acc + mrow[pl.ds(start + i * LANES, LANES)]
            acc = lax.fori_loop(0, count // LANES, sbody,
                                jnp.zeros((LANES,), jnp.int32))
            s = acc[0]
            for i in range(1, LANES):
                s = s + acc[i]
            return s

        q = msum(0, MAXQ)
        t = msum(MAXQ, MAXT)
        c = msum(MAXQ + MAXT, MAXC)
        svec = jnp.where(lane == 0, jnp.full((LANES,), q, jnp.int32),
                         jnp.where(lane == 1,
                                   jnp.full((LANES,), t, jnp.int32),
                                   jnp.full((LANES,), c, jnp.int32)))
        svbuf[pl.ds(0, LANES)] = svec
        pltpu.sync_copy(svbuf, shared.at[pl.ds(sid * LANES, LANES)])

    def zb(j, _):
        zbuf[j // (D // LANES), pl.ds((j % (D // LANES)) * LANES, LANES)] = zvec
        return 0
    lax.fori_loop(0, (CHUNK * D) // LANES, zb, 0)

    plsc.subcore_barrier()
    pltpu.sync_copy(shared, sumsbuf)

    def chunk_info(j):
        g128 = sid + NSUB * j
        bl = g128 // (L2 // CHUNK)
        g = g128 - bl * (L2 // CHUNK)
        svec = sumsbuf[pl.ds(bl * LANES, LANES)]
        q, t, c = svec[0], svec[1], svec[2]
        in_t = jnp.logical_and(g >= QCH, g < QCH + TCH)
        in_c = g >= QCH + TCH
        gg = jnp.where(in_c, g - QCH - TCH, jnp.where(in_t, g - QCH, g))
        n = jnp.where(in_c, c, jnp.where(in_t, t, q))
        src_row = jnp.where(in_c, q + t, jnp.where(in_t, q, 0))
        nv = jnp.minimum(n - gg * CHUNK, CHUNK)
        bex = cid * EPC + bl
        src = src_row + gg * CHUNK
        valid = nv > 0
        direct = jnp.logical_and(valid,
                                 jnp.logical_and(nv == CHUNK, src % 8 == 0))
        wstart = _mult8(jnp.minimum(src - src % 8, L1 - WIN))
        wref = inputs_hbm.at[bex, pl.ds(wstart, WIN), :]
        drd = inputs_hbm.at[bex, pl.ds(_mult8(src), CHUNK), :]
        dref = out_hbm.at[bex, pl.ds(_mult8(g * CHUNK), CHUNK), :]
        return valid, direct, wref, drd, src - wstart, nv, dref

    def shift_chunk(src_buf, sbase, nvalid, dst_buf):
        def crow(j, _):
            for ci in range(D // LANES):
                dst_buf[j, pl.ds(ci * LANES, LANES)] = (
                    src_buf[sbase + j, pl.ds(ci * LANES, LANES)])
            return 0
        lax.fori_loop(0, nvalid, crow, 0)

        def zrow(j, _):
            for ci in range(D // LANES):
                dst_buf[j, pl.ds(ci * LANES, LANES)] = zvec
            return 0
        lax.fori_loop(nvalid, CHUNK, zrow, 0)

    rslot0 = ring0.at[pl.ds(0, CHUNK), :]
    rslot1 = ring1.at[pl.ds(0, CHUNK), :]

    def proc(j, ring, rslot, obuf, rsem, wsem, prev_direct, first):
        valid, direct, wref, drd, sbase, nv, dref = chunk_info(j)

        @pl.when(jnp.logical_and(jnp.logical_not(first),
                                 jnp.logical_not(prev_direct)))
        def _():
            pltpu.make_async_copy(obuf, dref, wsem).wait()

        @pl.when(direct)
        def _():
            pltpu.make_async_copy(drd, rslot, rsem).wait()
            pltpu.make_async_copy(rslot, dref, wsem).start()

        @pl.when(jnp.logical_and(valid, jnp.logical_not(direct)))
        def _():
            pltpu.make_async_copy(wref, ring, rsem).wait()
            shift_chunk(ring, sbase, nv, obuf)
            pltpu.make_async_copy(obuf, dref, wsem).start()

        @pl.when(jnp.logical_not(valid))
        def _():
            pltpu.make_async_copy(zbuf, dref, wsem).start()

        return direct

    def prefetch(j, ring, rslot, rsem):
        valid, direct, wref, drd, _, _, _ = chunk_info(j)

        @pl.when(direct)
        def _():
            pltpu.make_async_copy(drd, rslot, rsem).start()

        @pl.when(jnp.logical_and(valid, jnp.logical_not(direct)))
        def _():
            pltpu.make_async_copy(wref, ring, rsem).start()

    last = out_hbm.at[cid * EPC, pl.ds(0, CHUNK), :]

    prefetch(0, ring0, rslot0, rsem0)

    def pair(p, carry):
        d0, d1 = carry
        j0 = 2 * p
        j1 = j0 + 1

        @pl.when(d1)
        def _():
            pltpu.make_async_copy(obuf1, last, wsem1).wait()
        prefetch(j1, ring1, rslot1, rsem1)

        nd0 = proc(j0, ring0, rslot0, obuf0, rsem0, wsem0, d0, p == 0)

        @pl.when(j0 + 2 < CPT)
        def _():
            @pl.when(nd0)
            def _():
                pltpu.make_async_copy(obuf0, last, wsem0).wait()
            prefetch(j0 + 2, ring0, rslot0, rsem0)

        nd1 = proc(j1, ring1, rslot1, obuf1, rsem1, wsem1, d1, p == 0)
        return nd0, nd1
    dlast0, dlast1 = lax.fori_loop(
        0, CPT // 2, pair,
        (jnp.bool_(False), jnp.bool_(False)))

    pltpu.make_async_copy(obuf0, last, wsem0).wait()
    pltpu.make_async_copy(obuf1, last, wsem1).wait()
    del dlast0, dlast1


@jax.jit
def _repack(inputs, maski):
    mesh = plsc.VectorSubcoreMesh(core_axis_name="c", subcore_axis_name="s")
    k = functools.partial(
        pl.kernel,
        mesh=mesh,
        out_type=jax.ShapeDtypeStruct((B, L2, D), jnp.float32),
        scratch_types=[
            pltpu.VMEM((L2,), jnp.int32),
            pltpu.VMEM((WIN, D), jnp.float32),
            pltpu.VMEM((WIN, D), jnp.float32),
            pltpu.VMEM((CHUNK, D), jnp.float32),
            pltpu.VMEM((CHUNK, D), jnp.float32),
            pltpu.VMEM((CHUNK, D), jnp.float32),
            pltpu.VMEM((LANES,), jnp.int32),
            pltpu.VMEM((EPC * LANES,), jnp.int32),
            pltpu.VMEM_SHARED((EPC * LANES,), jnp.int32),
            pltpu.SemaphoreType.DMA,
            pltpu.SemaphoreType.DMA,
            pltpu.SemaphoreType.DMA,
            pltpu.SemaphoreType.DMA,
            pltpu.SemaphoreType.DMA,
        ],
    )(_repack_body)
    return k(inputs, maski)


def kernel(inputs, mask, mask_split):
    del mask
    outputs = _repack(inputs, mask_split.astype(jnp.int32).reshape(-1))
    return outputs, mask_split
